# Initial kernel scaffold; baseline (speedup 1.0000x reference)
import jax
import jax.numpy as jnp
from jax.experimental import pallas as pl


def kernel(task_ids, module_logits):
    raise NotImplementedError("write your pallas kernel here")


# speedup vs baseline: 1.0554x; 1.0554x over previous
import functools

import jax
import jax.numpy as jnp
from jax import lax
from jax.experimental import pallas as pl
from jax.experimental.pallas import tpu as pltpu
from jax.experimental.pallas import tpu_sc as plsc

_EPS = 1e-12
_N_SPLITS = 8
_N_SKILLS = 16
_B = 16384
_D = _N_SPLITS * _N_SKILLS
_NW = 32
_B_PER_W = _B // _NW

_mesh = plsc.VectorSubcoreMesh(core_axis_name="c", subcore_axis_name="s")


@functools.partial(
    pl.kernel,
    out_type=jax.ShapeDtypeStruct((_B, _D), jnp.float32),
    mesh=_mesh,
    scratch_types=[
        pltpu.VMEM((_B_PER_W,), jnp.int32),
        pltpu.VMEM((_B_PER_W, _D), jnp.float32),
        pltpu.SemaphoreType.DMA,
    ],
    compiler_params=pltpu.CompilerParams(needs_layout_passes=False),
)
def _poly_select(ids_hbm, table_hbm, out_hbm, idx_v, rows_v, sem):
    wid = lax.axis_index("s") * 2 + lax.axis_index("c")
    base = wid * _B_PER_W
    pltpu.sync_copy(ids_hbm.at[pl.ds(base, _B_PER_W)], idx_v)
    pltpu.async_copy(table_hbm.at[idx_v], rows_v, sem).wait()

    @plsc.parallel_loop(0, _B_PER_W, 1, unroll=2)
    def _row(r):
        for s in range(_N_SPLITS):
            x = rows_v[r, pl.ds(s * _N_SKILLS, _N_SKILLS)]
            sig = 1.0 / (1.0 + jnp.exp(-x))
            tot = jnp.broadcast_to(jnp.sum(sig) + _EPS, (_N_SKILLS,))
            rows_v[r, pl.ds(s * _N_SKILLS, _N_SKILLS)] = sig / tot

    pltpu.sync_copy(rows_v, out_hbm.at[pl.ds(base, _B_PER_W)])


def kernel(task_ids, module_logits):
    out = _poly_select(task_ids.astype(jnp.int32), module_logits)
    return out.reshape(_B, _N_SPLITS, _N_SKILLS)
